# Initial kernel scaffold; baseline (speedup 1.0000x reference)
import jax
import jax.numpy as jnp
from jax.experimental import pallas as pl


def kernel(x, edge_index, edge_weight, W1, B1, GW0, GB0, GW1, GB1, W2, B2):
    raise NotImplementedError("write your pallas kernel here")


# speedup vs baseline: 4.5245x; 4.5245x over previous
import dataclasses
import functools

import jax
import jax.numpy as jnp
from jax import lax
from jax.experimental import pallas as pl
from jax.experimental.pallas import tpu as pltpu
from jax.experimental.pallas import tpu_sc as plsc

_NC, _NS, _L = 2, 16, 16
_C = 128
_NP = 10240
_RPT = _NP // _NS

_HIGH = lax.Precision.HIGHEST


def _make_sc_agg(n_nodes, n_edges, d):
  chunks = n_edges // _C
  cps = chunks // _NC
  mesh = plsc.VectorSubcoreMesh(core_axis_name="c", subcore_axis_name="s")
  cp = pltpu.CompilerParams()
  if "needs_layout_passes" in pltpu.CompilerParams.__dataclass_fields__:
    cp = dataclasses.replace(cp, needs_layout_passes=False)
  if d % 128 != 0:
    cp = dataclasses.replace(cp, use_tc_tiling_on_sc=False)

  @functools.partial(
      pl.kernel,
      compiler_params=cp,
      out_type=jax.ShapeDtypeStruct((_NC, _NP, d), jnp.float32),
      mesh=mesh,
      scratch_types=[
          pltpu.VMEM((_C,), jnp.int32),
          pltpu.VMEM((_C,), jnp.int32),
          pltpu.VMEM((_C,), jnp.float32),
          pltpu.VMEM((_C, d), jnp.float32),
          pltpu.VMEM_SHARED((_NP, d), jnp.float32),
      ],
  )
  def agg(table_hbm, col_hbm, row_hbm, w_hbm, out_hbm,
          colv, rowv, wv, rows, acc):
    cid = lax.axis_index("c")
    sid = lax.axis_index("s")

    @pl.loop(0, _C)
    def _zrow(i):
      for j in range(d // _L):
        rows[i, pl.ds(j * _L, _L)] = jnp.zeros((_L,), jnp.float32)

    base = sid * _RPT
    @pl.loop(0, _RPT // _C)
    def _zcp(b):
      pltpu.sync_copy(rows, acc.at[pl.ds(base + b * _C, _C)])
    plsc.subcore_barrier()

    n_my = (cps - sid + _NS - 1) // _NS
    @pl.loop(0, n_my)
    def _chunk(k):
      e0 = (cid * cps + sid + k * _NS) * _C
      pltpu.sync_copy(col_hbm.at[pl.ds(e0, _C)], colv)
      pltpu.sync_copy(row_hbm.at[pl.ds(e0, _C)], rowv)
      pltpu.sync_copy(w_hbm.at[pl.ds(e0, _C)], wv)
      pltpu.sync_copy(table_hbm.at[colv], rows)

      @pl.loop(0, _C)
      def _scale(i):
        w16 = plsc.load_gather(wv, [jnp.full((_L,), i, jnp.int32)])
        for j in range(d // _L):
          sl = pl.ds(j * _L, _L)
          rows[i, sl] = rows[i, sl] * w16

      pltpu.sync_copy(rows, acc.at[rowv], add=True)
    plsc.subcore_barrier()

    pltpu.sync_copy(acc.at[pl.ds(base, _RPT)],
                    out_hbm.at[cid].at[pl.ds(base, _RPT)])

  return agg


def _dense_a(x_ref, w1_ref, b1_ref, gw0_ref, o_ref):
  h = jnp.dot(x_ref[...], w1_ref[...], precision=_HIGH,
              preferred_element_type=jnp.float32)
  h = jnp.maximum(h + b1_ref[...], 0.0)
  o_ref[...] = jnp.dot(h, gw0_ref[...], precision=_HIGH,
                       preferred_element_type=jnp.float32)


def _dense_b(p_ref, b_ref, w_ref, o_ref):
  t = jnp.maximum(p_ref[0] + p_ref[1] + b_ref[...], 0.0)
  o_ref[...] = jnp.dot(t, w_ref[...], precision=_HIGH,
                       preferred_element_type=jnp.float32)


def _dense_c(p_ref, b_ref, w_ref, b2_ref, o_ref):
  t = jnp.maximum(p_ref[0] + p_ref[1] + b_ref[...], 0.0)
  t = jnp.dot(t, w_ref[...], precision=_HIGH,
              preferred_element_type=jnp.float32)
  o_ref[...] = jnp.maximum(t + b2_ref[...], 0.0)


def kernel(x, edge_index, edge_weight, W1, B1, GW0, GB0, GW1, GB1, W2, B2):
  n, d_in = x.shape
  row = edge_index[0]
  col = edge_index[1]

  h0 = W1.shape[1]
  h1 = GW0.shape[1]
  h2 = GW1.shape[1]
  d_out = W2.shape[1]

  blk = 1000
  xw0 = pl.pallas_call(
      _dense_a,
      grid=(n // blk,),
      in_specs=[
          pl.BlockSpec((blk, d_in), lambda i: (i, 0)),
          pl.BlockSpec((d_in, h0), lambda i: (0, 0)),
          pl.BlockSpec((1, h0), lambda i: (0, 0)),
          pl.BlockSpec((h0, h1), lambda i: (0, 0)),
      ],
      out_specs=pl.BlockSpec((blk, h1), lambda i: (i, 0)),
      out_shape=jax.ShapeDtypeStruct((n, h1), jnp.float32),
  )(x, W1, B1.reshape(1, -1), GW0)

  p0 = _make_sc_agg(n, row.shape[0], h1)(xw0, col, row, edge_weight)

  blkp = 1024
  xw1 = pl.pallas_call(
      _dense_b,
      grid=(_NP // blkp,),
      in_specs=[
          pl.BlockSpec((2, blkp, h1), lambda i: (0, i, 0)),
          pl.BlockSpec((1, h1), lambda i: (0, 0)),
          pl.BlockSpec((h1, h2), lambda i: (0, 0)),
      ],
      out_specs=pl.BlockSpec((blkp, h2), lambda i: (i, 0)),
      out_shape=jax.ShapeDtypeStruct((_NP, h2), jnp.float32),
  )(p0, GB0.reshape(1, -1), GW1)

  p1 = _make_sc_agg(n, row.shape[0], h2)(xw1, col, row, edge_weight)

  out = pl.pallas_call(
      _dense_c,
      grid=(_NP // blkp,),
      in_specs=[
          pl.BlockSpec((2, blkp, h2), lambda i: (0, i, 0)),
          pl.BlockSpec((1, h2), lambda i: (0, 0)),
          pl.BlockSpec((h2, d_out), lambda i: (0, 0)),
          pl.BlockSpec((1, d_out), lambda i: (0, 0)),
      ],
      out_specs=pl.BlockSpec((blkp, d_out), lambda i: (i, 0)),
      out_shape=jax.ShapeDtypeStruct((_NP, d_out), jnp.float32),
  )(p1, GB1.reshape(1, -1), W2, B2.reshape(1, -1))

  return out[:n]


# speedup vs baseline: 4.6253x; 1.0223x over previous
import dataclasses
import functools

import jax
import jax.numpy as jnp
from jax import lax
from jax.experimental import pallas as pl
from jax.experimental.pallas import tpu as pltpu
from jax.experimental.pallas import tpu_sc as plsc

_NC, _NS, _L = 2, 16, 16
_C = 64
_NP = 10240
_RPT = _NP // _NS

_HIGH = lax.Precision.HIGHEST


def _make_sc_agg(n_nodes, n_chunks, d):
  cpt = n_chunks // (_NC * _NS)
  mesh = plsc.VectorSubcoreMesh(core_axis_name="c", subcore_axis_name="s")
  cp = pltpu.CompilerParams()
  if "needs_layout_passes" in pltpu.CompilerParams.__dataclass_fields__:
    cp = dataclasses.replace(cp, needs_layout_passes=False)
  cp = dataclasses.replace(cp, use_tc_tiling_on_sc=False)

  @functools.partial(
      pl.kernel,
      compiler_params=cp,
      out_type=jax.ShapeDtypeStruct((_NC, _NP, d), jnp.float32),
      mesh=mesh,
      scratch_types=[
          pltpu.VMEM((cpt, _C), jnp.int32),
          pltpu.VMEM((cpt, _C), jnp.int32),
          pltpu.VMEM((cpt, _C), jnp.float32),
          pltpu.VMEM((_C, d), jnp.float32),
          pltpu.VMEM((_C, d), jnp.float32),
          pltpu.VMEM_SHARED((_NP, d), jnp.float32),
          pltpu.SemaphoreType.DMA,
          pltpu.SemaphoreType.DMA,
          pltpu.SemaphoreType.DMA,
          pltpu.SemaphoreType.DMA,
      ],
  )
  def agg(table_hbm, col_hbm, row_hbm, w_hbm, out_hbm,
          colb, rowb, wb, rows0, rows1, acc, gs0, gs1, ss0, ss1):
    cid = lax.axis_index("c")
    sid = lax.axis_index("s")
    rows = (rows0, rows1)
    gsem = (gs0, gs1)
    ssem = (ss0, ss1)

    @pl.loop(0, _C)
    def _zrow(i):
      for j in range(d // _L):
        rows0[i, pl.ds(j * _L, _L)] = jnp.zeros((_L,), jnp.float32)

    base = sid * _RPT
    @pl.loop(0, _RPT // _C)
    def _zcp(b):
      pltpu.sync_copy(rows0, acc.at[pl.ds(base + b * _C, _C)])
    plsc.subcore_barrier()

    c0 = (cid * _NS + sid) * cpt
    pltpu.sync_copy(col_hbm.at[pl.ds(c0, cpt)], colb)
    pltpu.sync_copy(row_hbm.at[pl.ds(c0, cpt)], rowb)
    pltpu.sync_copy(w_hbm.at[pl.ds(c0, cpt)], wb)
    pltpu.async_copy(table_hbm.at[colb.at[0]], rows0, gs0)
    pltpu.async_copy(table_hbm.at[colb.at[1]], rows1, gs1)

    @pl.loop(0, cpt // 2)
    def _pair(p):
      for b in range(2):
        k = 2 * p + b
        rb, gb, sb = rows[b], gsem[b], ssem[b]
        pltpu.make_async_copy(table_hbm.at[colb.at[k]], rb, gb).wait()

        k16 = jnp.full((_L,), k, jnp.int32)
        @plsc.parallel_loop(0, _C, unroll=4)
        def _scale(i):
          w16 = plsc.load_gather(wb, [k16, jnp.full((_L,), i, jnp.int32)])
          for j in range(d // _L):
            sl = pl.ds(j * _L, _L)
            rb[i, sl] = rb[i, sl] * w16

        pltpu.async_copy(rb, acc.at[rowb.at[k]], sb, add=True)

        @pl.when(p < cpt // 2 - 1)
        def _prefetch():
          pltpu.make_async_copy(rb, acc.at[rowb.at[k]], sb).wait()
          pltpu.async_copy(table_hbm.at[colb.at[k + 2]], rb, gb)

    pltpu.make_async_copy(rows0, acc.at[rowb.at[cpt - 2]], ss0).wait()
    pltpu.make_async_copy(rows1, acc.at[rowb.at[cpt - 1]], ss1).wait()
    plsc.subcore_barrier()

    pltpu.sync_copy(acc.at[pl.ds(base, _RPT)],
                    out_hbm.at[cid].at[pl.ds(base, _RPT)])

  return agg


def _dense_a(x_ref, w1_ref, b1_ref, gw0_ref, o_ref):
  h = jnp.dot(x_ref[...], w1_ref[...], precision=_HIGH,
              preferred_element_type=jnp.float32)
  h = jnp.maximum(h + b1_ref[...], 0.0)
  o_ref[...] = jnp.dot(h, gw0_ref[...], precision=_HIGH,
                       preferred_element_type=jnp.float32)


def _dense_b(p_ref, b_ref, w_ref, o_ref):
  t = jnp.maximum(p_ref[0] + p_ref[1] + b_ref[...], 0.0)
  o_ref[...] = jnp.dot(t, w_ref[...], precision=_HIGH,
                       preferred_element_type=jnp.float32)


def _dense_c(p_ref, b_ref, w_ref, b2_ref, o_ref):
  t = jnp.maximum(p_ref[0] + p_ref[1] + b_ref[...], 0.0)
  t = jnp.dot(t, w_ref[...], precision=_HIGH,
              preferred_element_type=jnp.float32)
  o_ref[...] = jnp.maximum(t + b2_ref[...], 0.0)


def kernel(x, edge_index, edge_weight, W1, B1, GW0, GB0, GW1, GB1, W2, B2):
  n, d_in = x.shape
  e = edge_weight.shape[0]

  cpt = -(-e // (_C * _NC * _NS * 8)) * 8
  n_chunks = cpt * _NC * _NS
  pad = n_chunks * _C - e
  col2 = jnp.pad(edge_index[1], (0, pad)).reshape(n_chunks, _C)
  row2 = jnp.pad(edge_index[0], (0, pad)).reshape(n_chunks, _C)
  w2 = jnp.pad(edge_weight, (0, pad)).reshape(n_chunks, _C)

  h0 = W1.shape[1]
  h1 = GW0.shape[1]
  h2 = GW1.shape[1]
  d_out = W2.shape[1]

  blk = 1000
  xw0 = pl.pallas_call(
      _dense_a,
      grid=(n // blk,),
      in_specs=[
          pl.BlockSpec((blk, d_in), lambda i: (i, 0)),
          pl.BlockSpec((d_in, h0), lambda i: (0, 0)),
          pl.BlockSpec((1, h0), lambda i: (0, 0)),
          pl.BlockSpec((h0, h1), lambda i: (0, 0)),
      ],
      out_specs=pl.BlockSpec((blk, h1), lambda i: (i, 0)),
      out_shape=jax.ShapeDtypeStruct((n, h1), jnp.float32),
  )(x, W1, B1.reshape(1, -1), GW0)

  p0 = _make_sc_agg(n, n_chunks, h1)(xw0, col2, row2, w2)

  blkp = 1024
  xw1 = pl.pallas_call(
      _dense_b,
      grid=(_NP // blkp,),
      in_specs=[
          pl.BlockSpec((2, blkp, h1), lambda i: (0, i, 0)),
          pl.BlockSpec((1, h1), lambda i: (0, 0)),
          pl.BlockSpec((h1, h2), lambda i: (0, 0)),
      ],
      out_specs=pl.BlockSpec((blkp, h2), lambda i: (i, 0)),
      out_shape=jax.ShapeDtypeStruct((_NP, h2), jnp.float32),
  )(p0, GB0.reshape(1, -1), GW1)

  p1 = _make_sc_agg(n, n_chunks, h2)(xw1, col2, row2, w2)

  out = pl.pallas_call(
      _dense_c,
      grid=(_NP // blkp,),
      in_specs=[
          pl.BlockSpec((2, blkp, h2), lambda i: (0, i, 0)),
          pl.BlockSpec((1, h2), lambda i: (0, 0)),
          pl.BlockSpec((h2, d_out), lambda i: (0, 0)),
          pl.BlockSpec((1, d_out), lambda i: (0, 0)),
      ],
      out_specs=pl.BlockSpec((blkp, d_out), lambda i: (i, 0)),
      out_shape=jax.ShapeDtypeStruct((_NP, d_out), jnp.float32),
  )(p1, GB1.reshape(1, -1), W2, B2.reshape(1, -1))

  return out[:n]
